# Initial kernel scaffold; baseline (speedup 1.0000x reference)
import jax
import jax.numpy as jnp
from jax.experimental import pallas as pl


def kernel(x, edge_index, edge_attr, batch, params):
    raise NotImplementedError("write your pallas kernel here")


# speedup vs baseline: 1.4138x; 1.4138x over previous
import functools

import jax
import jax.numpy as jnp
from jax.experimental import pallas as pl


def _pad_axis(a, mult, axis):
    size = a.shape[axis]
    pad = (-size) % mult
    if pad == 0:
        return a
    widths = [(0, 0)] * a.ndim
    widths[axis] = (0, pad)
    return jnp.pad(a, widths)


def _mm_body(x_ref, w_ref, b_ref, o_ref, *, relu):
    acc = jnp.dot(x_ref[...], w_ref[...], preferred_element_type=jnp.float32)
    acc = acc + b_ref[...]
    if relu:
        acc = jnp.maximum(acc, 0.0)
    o_ref[...] = acc


def _mm(x, w, b, relu=False, block_m=512):
    m, n = x.shape[0], w.shape[1]
    xp = _pad_axis(_pad_axis(x, block_m, 0), 128, 1)
    wp = _pad_axis(_pad_axis(w, 128, 0), 128, 1)
    bp = _pad_axis(b.reshape(1, -1), 128, 1)
    mp, kp = xp.shape
    np_ = wp.shape[1]
    out = pl.pallas_call(
        functools.partial(_mm_body, relu=relu),
        grid=(mp // block_m,),
        in_specs=[
            pl.BlockSpec((block_m, kp), lambda i: (i, 0)),
            pl.BlockSpec((kp, np_), lambda i: (0, 0)),
            pl.BlockSpec((1, np_), lambda i: (0, 0)),
        ],
        out_specs=pl.BlockSpec((block_m, np_), lambda i: (i, 0)),
        out_shape=jax.ShapeDtypeStruct((mp, np_), jnp.float32),
    )(xp, wp, bp)
    return out[:m, :n]


def _mm2_body(x_ref, wx_ref, y_ref, wy_ref, b_ref, o_ref, *, relu):
    acc = jnp.dot(x_ref[...], wx_ref[...], preferred_element_type=jnp.float32)
    acc = acc + jnp.dot(y_ref[...], wy_ref[...], preferred_element_type=jnp.float32)
    acc = acc + b_ref[...]
    if relu:
        acc = jnp.maximum(acc, 0.0)
    o_ref[...] = acc


def _mm2(x, wx, y, wy, b, relu=False, block_m=512):
    m, n = x.shape[0], wx.shape[1]
    xp = _pad_axis(_pad_axis(x, block_m, 0), 128, 1)
    yp = _pad_axis(_pad_axis(y, block_m, 0), 128, 1)
    wxp = _pad_axis(_pad_axis(wx, 128, 0), 128, 1)
    wyp = _pad_axis(_pad_axis(wy, 128, 0), 128, 1)
    bp = _pad_axis(b.reshape(1, -1), 128, 1)
    mp, kxp = xp.shape
    kyp = yp.shape[1]
    np_ = wxp.shape[1]
    out = pl.pallas_call(
        functools.partial(_mm2_body, relu=relu),
        grid=(mp // block_m,),
        in_specs=[
            pl.BlockSpec((block_m, kxp), lambda i: (i, 0)),
            pl.BlockSpec((kxp, np_), lambda i: (0, 0)),
            pl.BlockSpec((block_m, kyp), lambda i: (i, 0)),
            pl.BlockSpec((kyp, np_), lambda i: (0, 0)),
            pl.BlockSpec((1, np_), lambda i: (0, 0)),
        ],
        out_specs=pl.BlockSpec((block_m, np_), lambda i: (i, 0)),
        out_shape=jax.ShapeDtypeStruct((mp, np_), jnp.float32),
    )(xp, wxp, yp, wyp, bp)
    return out[:m, :n]


def _edge_mlp_body(ad_ref, bs_ref, attr_ref, c_ref, w2_ref, b2_ref,
                   w3_ref, b3_ref, o_ref):
    attr = attr_ref[...]
    tbl = c_ref[...]
    iota = jax.lax.broadcasted_iota(jnp.int32, (attr.shape[0], tbl.shape[0]), 1)
    onehot = (attr == iota).astype(jnp.float32)
    ca = jnp.dot(onehot, tbl, preferred_element_type=jnp.float32)
    t1 = jnp.maximum(ad_ref[...] + bs_ref[...] + ca, 0.0)
    t2 = jnp.dot(t1, w2_ref[...], preferred_element_type=jnp.float32) + b2_ref[...]
    t2 = jnp.maximum(t2, 0.0)
    o_ref[...] = jnp.dot(t2, w3_ref[...], preferred_element_type=jnp.float32) + b3_ref[...]


def _edge_mlp(a_dst, b_src, attr, c_tbl, w2, b2, w3, b3, block_m=512):
    e, f = a_dst.shape
    ap = _pad_axis(_pad_axis(a_dst, block_m, 0), 128, 1)
    bp_ = _pad_axis(_pad_axis(b_src, block_m, 0), 128, 1)
    attrp = _pad_axis(attr.reshape(-1, 1), block_m, 0)
    cp = _pad_axis(_pad_axis(c_tbl, 128, 0), 128, 1)
    w2p = _pad_axis(_pad_axis(w2, 128, 0), 128, 1)
    w3p = _pad_axis(_pad_axis(w3, 128, 0), 128, 1)
    b2p = _pad_axis(b2.reshape(1, -1), 128, 1)
    b3p = _pad_axis(b3.reshape(1, -1), 128, 1)
    ep, fp = ap.shape
    out = pl.pallas_call(
        _edge_mlp_body,
        grid=(ep // block_m,),
        in_specs=[
            pl.BlockSpec((block_m, fp), lambda i: (i, 0)),
            pl.BlockSpec((block_m, fp), lambda i: (i, 0)),
            pl.BlockSpec((block_m, 1), lambda i: (i, 0)),
            pl.BlockSpec((cp.shape[0], fp), lambda i: (0, 0)),
            pl.BlockSpec((fp, fp), lambda i: (0, 0)),
            pl.BlockSpec((1, fp), lambda i: (0, 0)),
            pl.BlockSpec((fp, fp), lambda i: (0, 0)),
            pl.BlockSpec((1, fp), lambda i: (0, 0)),
        ],
        out_specs=pl.BlockSpec((block_m, fp), lambda i: (i, 0)),
        out_shape=jax.ShapeDtypeStruct((ep, fp), jnp.float32),
    )(ap, bp_, attrp, cp, w2p, b2p, w3p, b3p)
    return out[:e, :f]


def kernel(x, edge_index, edge_attr, batch, params):
    src, dst = edge_index[0], edge_index[1]
    n_nodes = x.shape[0]
    n_graphs = 16
    e_emb_tbl = params['edge_emb']
    agg_w = jax.nn.softmax(params['agg_w'])

    w0, b0 = params['pre_mlp']
    h = _mm(x, w0, b0, relu=True)

    for ck in ('conv1', 'conv2'):
        p = params[ck]
        f_in = h.shape[1]
        we, be = p['edge_enc']
        (w1, b1), (w2, b2), (w3, b3) = p['pre']
        w1d, w1s, w1e = w1[:f_in], w1[f_in:2 * f_in], w1[2 * f_in:]
        zeros = jnp.zeros((w1.shape[1],), jnp.float32)
        a = _mm(h, w1d, zeros)
        b_ = _mm(h, w1s, zeros)
        c_tbl = (e_emb_tbl @ we + be) @ w1e + b1
        a_dst = jnp.take(a, dst, axis=0)
        b_src = jnp.take(b_, src, axis=0)
        hm = _edge_mlp(a_dst, b_src, edge_attr, c_tbl, w2, b2, w3, b3)

        s = jax.ops.segment_sum(hm, dst, num_segments=n_nodes)
        cnt = jax.ops.segment_sum(jnp.ones((hm.shape[0],), hm.dtype), dst,
                                  num_segments=n_nodes)
        cnt_c = jnp.maximum(cnt, 1.0)[:, None]
        mean = s / cnt_c
        mn = jax.ops.segment_min(hm, dst, num_segments=n_nodes)
        mx = jax.ops.segment_max(hm, dst, num_segments=n_nodes)
        mn = jnp.where(jnp.isfinite(mn), mn, 0.0)
        mx = jnp.where(jnp.isfinite(mx), mx, 0.0)
        sq = jax.ops.segment_sum(hm * hm, dst, num_segments=n_nodes)
        var = jnp.maximum(sq / cnt_c - mean * mean, 0.0)
        std = jnp.sqrt(var + 1e-5)
        agg = (agg_w[0] * s + agg_w[1] * mean + agg_w[2] * mn
               + agg_w[3] * mx + agg_w[4] * std)

        (wp1, bp1), (wp2, bp2), (wp3, bp3) = p['post']
        wpx, wpa = wp1[:f_in], wp1[f_in:]
        out = _mm2(h, wpx, agg, wpa, bp1, relu=True)
        out = _mm(out, wp2, bp2, relu=True)
        out = _mm(out, wp3, bp3)
        wl, bl = p['lin']
        out = _mm(out, wl, bl)

        gamma, beta = p['bn']
        mu = out.mean(axis=0)
        var_bn = out.var(axis=0)
        h = jax.nn.relu((out - mu) / jnp.sqrt(var_bn + 1e-5) * gamma + beta)

    pooled = jax.ops.segment_sum(h, batch, num_segments=n_graphs)

    (we1, be1), (we2, be2), (we3, be3) = params['ep']
    out = _mm(pooled, we1, be1, relu=True, block_m=16)
    out = _mm(out, we2, be2, relu=True, block_m=16)
    out = _mm(out, we3, be3, block_m=16)
    return out


# speedup vs baseline: 1.4278x; 1.0099x over previous
import functools

import jax
import jax.numpy as jnp
from jax.experimental import pallas as pl


def _pad_axis(a, mult, axis):
    size = a.shape[axis]
    pad = (-size) % mult
    if pad == 0:
        return a
    widths = [(0, 0)] * a.ndim
    widths[axis] = (0, pad)
    return jnp.pad(a, widths)


def _mm_body(x_ref, w_ref, b_ref, o_ref, *, relu):
    acc = jnp.dot(x_ref[...], w_ref[...], preferred_element_type=jnp.float32)
    acc = acc + b_ref[...]
    if relu:
        acc = jnp.maximum(acc, 0.0)
    o_ref[...] = acc


def _mm(x, w, b, relu=False, block_m=512):
    m, n = x.shape[0], w.shape[1]
    xp = _pad_axis(_pad_axis(x, block_m, 0), 128, 1)
    wp = _pad_axis(_pad_axis(w, 128, 0), 128, 1)
    bp = _pad_axis(b.reshape(1, -1), 128, 1)
    mp, kp = xp.shape
    np_ = wp.shape[1]
    out = pl.pallas_call(
        functools.partial(_mm_body, relu=relu),
        grid=(mp // block_m,),
        in_specs=[
            pl.BlockSpec((block_m, kp), lambda i: (i, 0)),
            pl.BlockSpec((kp, np_), lambda i: (0, 0)),
            pl.BlockSpec((1, np_), lambda i: (0, 0)),
        ],
        out_specs=pl.BlockSpec((block_m, np_), lambda i: (i, 0)),
        out_shape=jax.ShapeDtypeStruct((mp, np_), jnp.float32),
    )(xp, wp, bp)
    return out[:m, :n]


def _mm2_body(x_ref, wx_ref, y_ref, wy_ref, b_ref, o_ref, *, relu):
    acc = jnp.dot(x_ref[...], wx_ref[...], preferred_element_type=jnp.float32)
    acc = acc + jnp.dot(y_ref[...], wy_ref[...], preferred_element_type=jnp.float32)
    acc = acc + b_ref[...]
    if relu:
        acc = jnp.maximum(acc, 0.0)
    o_ref[...] = acc


def _mm2(x, wx, y, wy, b, relu=False, block_m=512):
    m, n = x.shape[0], wx.shape[1]
    xp = _pad_axis(_pad_axis(x, block_m, 0), 128, 1)
    yp = _pad_axis(_pad_axis(y, block_m, 0), 128, 1)
    wxp = _pad_axis(_pad_axis(wx, 128, 0), 128, 1)
    wyp = _pad_axis(_pad_axis(wy, 128, 0), 128, 1)
    bp = _pad_axis(b.reshape(1, -1), 128, 1)
    mp, kxp = xp.shape
    kyp = yp.shape[1]
    np_ = wxp.shape[1]
    out = pl.pallas_call(
        functools.partial(_mm2_body, relu=relu),
        grid=(mp // block_m,),
        in_specs=[
            pl.BlockSpec((block_m, kxp), lambda i: (i, 0)),
            pl.BlockSpec((kxp, np_), lambda i: (0, 0)),
            pl.BlockSpec((block_m, kyp), lambda i: (i, 0)),
            pl.BlockSpec((kyp, np_), lambda i: (0, 0)),
            pl.BlockSpec((1, np_), lambda i: (0, 0)),
        ],
        out_specs=pl.BlockSpec((block_m, np_), lambda i: (i, 0)),
        out_shape=jax.ShapeDtypeStruct((mp, np_), jnp.float32),
    )(xp, wxp, yp, wyp, bp)
    return out[:m, :n]


def _edge_mlp_body(ad_ref, bs_ref, attr_ref, c_ref, w2_ref, b2_ref,
                   w3_ref, b3_ref, o_ref):
    attr = attr_ref[...]
    tbl = c_ref[...]
    iota = jax.lax.broadcasted_iota(jnp.int32, (attr.shape[0], tbl.shape[0]), 1)
    onehot = (attr == iota).astype(jnp.float32)
    ca = jnp.dot(onehot, tbl, preferred_element_type=jnp.float32)
    t1 = jnp.maximum(ad_ref[...] + bs_ref[...] + ca, 0.0)
    t2 = jnp.dot(t1, w2_ref[...], preferred_element_type=jnp.float32) + b2_ref[...]
    t2 = jnp.maximum(t2, 0.0)
    o_ref[...] = jnp.dot(t2, w3_ref[...], preferred_element_type=jnp.float32) + b3_ref[...]


def _edge_mlp(a_dst, b_src, attr, c_tbl, w2, b2, w3, b3, block_m=1024):
    e, f = a_dst.shape
    ap = _pad_axis(_pad_axis(a_dst, block_m, 0), 128, 1)
    bp_ = _pad_axis(_pad_axis(b_src, block_m, 0), 128, 1)
    attrp = _pad_axis(attr.reshape(-1, 1), block_m, 0)
    cp = _pad_axis(_pad_axis(c_tbl, 128, 0), 128, 1)
    w2p = _pad_axis(_pad_axis(w2, 128, 0), 128, 1)
    w3p = _pad_axis(_pad_axis(w3, 128, 0), 128, 1)
    b2p = _pad_axis(b2.reshape(1, -1), 128, 1)
    b3p = _pad_axis(b3.reshape(1, -1), 128, 1)
    ep, fp = ap.shape
    out = pl.pallas_call(
        _edge_mlp_body,
        grid=(ep // block_m,),
        in_specs=[
            pl.BlockSpec((block_m, fp), lambda i: (i, 0)),
            pl.BlockSpec((block_m, fp), lambda i: (i, 0)),
            pl.BlockSpec((block_m, 1), lambda i: (i, 0)),
            pl.BlockSpec((cp.shape[0], fp), lambda i: (0, 0)),
            pl.BlockSpec((fp, fp), lambda i: (0, 0)),
            pl.BlockSpec((1, fp), lambda i: (0, 0)),
            pl.BlockSpec((fp, fp), lambda i: (0, 0)),
            pl.BlockSpec((1, fp), lambda i: (0, 0)),
        ],
        out_specs=pl.BlockSpec((block_m, fp), lambda i: (i, 0)),
        out_shape=jax.ShapeDtypeStruct((ep, fp), jnp.float32),
    )(ap, bp_, attrp, cp, w2p, b2p, w3p, b3p)
    return out[:e, :f]


def kernel(x, edge_index, edge_attr, batch, params):
    src, dst = edge_index[0], edge_index[1]
    n_nodes = x.shape[0]
    n_graphs = 16
    e_emb_tbl = params['edge_emb']
    agg_w = jax.nn.softmax(params['agg_w'])

    w0, b0 = params['pre_mlp']
    h = _mm(x, w0, b0, relu=True)

    for ck in ('conv1', 'conv2'):
        p = params[ck]
        f_in = h.shape[1]
        we, be = p['edge_enc']
        (w1, b1), (w2, b2), (w3, b3) = p['pre']
        w1d, w1s, w1e = w1[:f_in], w1[f_in:2 * f_in], w1[2 * f_in:]
        zeros = jnp.zeros((w1.shape[1],), jnp.float32)
        a = _mm(h, w1d, zeros)
        b_ = _mm(h, w1s, zeros)
        c_tbl = (e_emb_tbl @ we + be) @ w1e + b1
        a_dst = jnp.take(a, dst, axis=0)
        b_src = jnp.take(b_, src, axis=0)
        hm = _edge_mlp(a_dst, b_src, edge_attr, c_tbl, w2, b2, w3, b3)

        s = jax.ops.segment_sum(hm, dst, num_segments=n_nodes)
        cnt = jax.ops.segment_sum(jnp.ones((hm.shape[0],), hm.dtype), dst,
                                  num_segments=n_nodes)
        cnt_c = jnp.maximum(cnt, 1.0)[:, None]
        mean = s / cnt_c
        mn = jax.ops.segment_min(hm, dst, num_segments=n_nodes)
        mx = jax.ops.segment_max(hm, dst, num_segments=n_nodes)
        mn = jnp.where(jnp.isfinite(mn), mn, 0.0)
        mx = jnp.where(jnp.isfinite(mx), mx, 0.0)
        sq = jax.ops.segment_sum(hm * hm, dst, num_segments=n_nodes)
        var = jnp.maximum(sq / cnt_c - mean * mean, 0.0)
        std = jnp.sqrt(var + 1e-5)
        agg = (agg_w[0] * s + agg_w[1] * mean + agg_w[2] * mn
               + agg_w[3] * mx + agg_w[4] * std)

        (wp1, bp1), (wp2, bp2), (wp3, bp3) = p['post']
        wpx, wpa = wp1[:f_in], wp1[f_in:]
        out = _mm2(h, wpx, agg, wpa, bp1, relu=True)
        out = _mm(out, wp2, bp2, relu=True)
        out = _mm(out, wp3, bp3)
        wl, bl = p['lin']
        out = _mm(out, wl, bl)

        gamma, beta = p['bn']
        mu = out.mean(axis=0)
        var_bn = out.var(axis=0)
        h = jax.nn.relu((out - mu) / jnp.sqrt(var_bn + 1e-5) * gamma + beta)

    pooled = jax.ops.segment_sum(h, batch, num_segments=n_graphs)

    (we1, be1), (we2, be2), (we3, be3) = params['ep']
    out = _mm(pooled, we1, be1, relu=True, block_m=16)
    out = _mm(out, we2, be2, relu=True, block_m=16)
    out = _mm(out, we3, be3, block_m=16)
    return out
